# baseline (speedup 1.0000x reference)
import functools

import jax
import jax.numpy as jnp
from jax import lax
from jax.experimental import pallas as pl
from jax.experimental.pallas import tpu as pltpu
from jax.experimental.pallas import tpu_sc as plsc

VOCAB = 1000000
EMBED = 16
BATCH = 4096

_ROWS_PER_BLOCK = 128 // EMBED
_NBLK = VOCAB // _ROWS_PER_BLOCK

_NC = 2
_NS = 16
_NW = _NC * _NS
_BPW = BATCH // _NW
_L = 16


def _sc_gather_kernel():
    mesh = plsc.VectorSubcoreMesh(core_axis_name="c", subcore_axis_name="s")

    @functools.partial(
        pl.kernel,
        mesh=mesh,
        compiler_params=pltpu.CompilerParams(
            use_tc_tiling_on_sc=False, needs_layout_passes=False),
        out_type=(
            jax.ShapeDtypeStruct((BATCH, EMBED), jnp.float32),
            jax.ShapeDtypeStruct((BATCH, EMBED), jnp.float32),
        ),
        scratch_types=[
            pltpu.VMEM((_BPW,), jnp.int32),
            pltpu.VMEM((_BPW,), jnp.int32),
            pltpu.VMEM((_BPW, 128), jnp.float32),
            pltpu.VMEM((_BPW, EMBED), jnp.float32),
            pltpu.VMEM((_BPW,), jnp.int32),
            pltpu.VMEM((_BPW,), jnp.int32),
            pltpu.VMEM((_BPW, 128), jnp.float32),
            pltpu.VMEM((_BPW, EMBED), jnp.float32),
            pltpu.SemaphoreType.DMA,
            pltpu.SemaphoreType.DMA,
        ],
    )
    def gather(cw_hbm, xw_hbm, vt_hbm, ut_hbm, outv_hbm, outu_hbm,
               idx_c, blk_c, rows_c, out_c, idx_x, blk_x, rows_x, out_x,
               sem_c, sem_x):
        wid = lax.axis_index("s") * _NC + lax.axis_index("c")
        base = wid * _BPW
        pltpu.sync_copy(cw_hbm.at[pl.ds(base, _BPW)], idx_c)
        pltpu.sync_copy(xw_hbm.at[pl.ds(base, _BPW)], idx_x)
        for t in range(_BPW // _L):
            sl = pl.ds(t * _L, _L)
            blk_c[sl] = idx_c[sl] >> 3
            blk_x[sl] = idx_x[sl] >> 3
        cp_c = pltpu.async_copy(vt_hbm.at[blk_c], rows_c, sem_c)
        cp_x = pltpu.async_copy(ut_hbm.at[blk_x], rows_x, sem_x)

        lanes = lax.iota(jnp.int32, _L)

        def extract(idx_ref, rows_ref, out_ref):
            for t in range(_BPW // _L):
                sl = pl.ds(t * _L, _L)
                col0 = (idx_ref[sl] & (_ROWS_PER_BLOCK - 1)) << 4
                rowv = t * _L + lanes
                for e in range(EMBED):
                    val = plsc.load_gather(rows_ref, [rowv, col0 + e])
                    plsc.store_scatter(
                        out_ref, [rowv, jnp.full((_L,), e, jnp.int32)], val)

        cp_c.wait()
        extract(idx_c, rows_c, out_c)
        pltpu.sync_copy(out_c, outv_hbm.at[pl.ds(base, _BPW)])
        cp_x.wait()
        extract(idx_x, rows_x, out_x)
        pltpu.sync_copy(out_x, outu_hbm.at[pl.ds(base, _BPW)])

    return gather


_ROW_TILE = 256


def _score_softmax_body(c_ref, x_ref, o_ref):
    scores = lax.dot_general(
        c_ref[...], x_ref[...],
        dimension_numbers=(((1,), (1,)), ((), ())),
        preferred_element_type=jnp.float32,
    )
    m = jnp.max(scores, axis=1, keepdims=True)
    e = jnp.exp(scores - m)
    s = jnp.sum(e, axis=1, keepdims=True)
    o_ref[...] = (scores - m) - jnp.log(s)


def kernel(center_words, context_words, embedding_v, embedding_u):
    center_embed, context_embed = _sc_gather_kernel()(
        center_words.astype(jnp.int32), context_words.astype(jnp.int32),
        embedding_v.reshape(_NBLK, 128), embedding_u.reshape(_NBLK, 128))

    log_probs = pl.pallas_call(
        _score_softmax_body,
        grid=(BATCH // _ROW_TILE,),
        in_specs=[
            pl.BlockSpec((_ROW_TILE, EMBED), lambda i: (i, 0)),
            pl.BlockSpec((BATCH, EMBED), lambda i: (0, 0)),
        ],
        out_specs=pl.BlockSpec((_ROW_TILE, BATCH), lambda i: (i, 0)),
        out_shape=jax.ShapeDtypeStruct((BATCH, BATCH), jnp.float32),
    )(center_embed, context_embed)
    return log_probs


# speedup vs baseline: 8.6401x; 8.6401x over previous
import functools

import jax
import jax.numpy as jnp
from jax import lax
from jax.experimental import pallas as pl
from jax.experimental.pallas import tpu as pltpu
from jax.experimental.pallas import tpu_sc as plsc

VOCAB = 1000000
EMBED = 16
BATCH = 4096

_NC = 2
_NS = 16
_NW = _NC * _NS
_BPW = BATCH // _NW
_L = 16
_GRP = _BPW // _L


def _sc_gather_kernel():
    mesh = plsc.VectorSubcoreMesh(core_axis_name="c", subcore_axis_name="s")

    @functools.partial(
        pl.kernel,
        mesh=mesh,
        compiler_params=pltpu.CompilerParams(needs_layout_passes=False),
        out_type=(
            jax.ShapeDtypeStruct((BATCH, 128), jnp.float32),
            jax.ShapeDtypeStruct((BATCH, 128), jnp.float32),
        ),
        scratch_types=[
            pltpu.VMEM((_BPW,), jnp.int32),
            pltpu.VMEM((_BPW,), jnp.int32),
            pltpu.VMEM((_L, 2, 8, 128), jnp.float32),
            pltpu.VMEM((_BPW, 128), jnp.float32),
            pltpu.VMEM((_BPW, 128), jnp.float32),
            pltpu.SemaphoreType.DMA,
        ],
    )
    def gather(cw_hbm, xw_hbm, vt_hbm, ut_hbm, outv_hbm, outu_hbm,
               idx_c, idx_x, buf, out_c, out_x, sem):
        wid = lax.axis_index("s") * _NC + lax.axis_index("c")
        base = wid * _BPW
        pltpu.sync_copy(cw_hbm.at[pl.ds(base, _BPW)], idx_c)
        pltpu.sync_copy(xw_hbm.at[pl.ds(base, _BPW)], idx_x)

        lanes = lax.iota(jnp.int32, _L)

        def one_table(table_hbm, idx, out):
            for g in range(_GRP):
                vchunk = idx[pl.ds(g * _L, _L)] >> 7
                cps = []
                for k in range(_L):
                    c = jnp.max(jnp.where(lanes == k, vchunk, 0))
                    start = pl.multiple_of(c * 128, 128)
                    cps.append(pltpu.async_copy(
                        table_hbm.at[:, :, pl.ds(start, 128)],
                        buf.at[k], sem))
                for cp in cps:
                    cp.wait()
                lvec = idx[pl.ds(g * _L, _L)] & 127
                rowv = g * _L + lanes
                for e in range(EMBED):
                    val = plsc.load_gather(
                        buf,
                        [lanes, jnp.full((_L,), e // 8, jnp.int32),
                         jnp.full((_L,), e % 8, jnp.int32), lvec])
                    plsc.store_scatter(
                        out, [rowv, jnp.full((_L,), e, jnp.int32)], val)

        one_table(vt_hbm, idx_c, out_c)
        pltpu.sync_copy(out_c, outv_hbm.at[pl.ds(base, _BPW)])
        one_table(ut_hbm, idx_x, out_x)
        pltpu.sync_copy(out_x, outu_hbm.at[pl.ds(base, _BPW)])

    return gather


_ROW_TILE = 256


def _score_softmax_body(c_ref, x_ref, o_ref):
    scores = lax.dot_general(
        c_ref[:, :EMBED], x_ref[:, :EMBED],
        dimension_numbers=(((1,), (1,)), ((), ())),
        preferred_element_type=jnp.float32,
    )
    m = jnp.max(scores, axis=1, keepdims=True)
    e = jnp.exp(scores - m)
    s = jnp.sum(e, axis=1, keepdims=True)
    o_ref[...] = (scores - m) - jnp.log(s)


def kernel(center_words, context_words, embedding_v, embedding_u):
    vt = embedding_v.T.reshape(2, 8, VOCAB)
    ut = embedding_u.T.reshape(2, 8, VOCAB)
    center_embed, context_embed = _sc_gather_kernel()(
        center_words.astype(jnp.int32), context_words.astype(jnp.int32),
        vt, ut)

    log_probs = pl.pallas_call(
        _score_softmax_body,
        grid=(BATCH // _ROW_TILE,),
        in_specs=[
            pl.BlockSpec((_ROW_TILE, 128), lambda i: (i, 0)),
            pl.BlockSpec((BATCH, 128), lambda i: (0, 0)),
        ],
        out_specs=pl.BlockSpec((_ROW_TILE, BATCH), lambda i: (i, 0)),
        out_shape=jax.ShapeDtypeStruct((BATCH, BATCH), jnp.float32),
    )(center_embed, context_embed)
    return log_probs


# speedup vs baseline: 9.5636x; 1.1069x over previous
import functools

import jax
import jax.numpy as jnp
from jax import lax
from jax.experimental import pallas as pl
from jax.experimental.pallas import tpu as pltpu
from jax.experimental.pallas import tpu_sc as plsc

VOCAB = 1000000
EMBED = 16
BATCH = 4096

_NC = 2
_NS = 16
_NW = _NC * _NS
_BPW = BATCH // _NW
_L = 16
_GRP = _BPW // _L


def _sc_gather_kernel():
    mesh = plsc.VectorSubcoreMesh(core_axis_name="c", subcore_axis_name="s")

    @functools.partial(
        pl.kernel,
        mesh=mesh,
        compiler_params=pltpu.CompilerParams(needs_layout_passes=False),
        out_type=(
            jax.ShapeDtypeStruct((BATCH, 128), jnp.float32),
            jax.ShapeDtypeStruct((BATCH, 128), jnp.float32),
        ),
        scratch_types=[
            pltpu.VMEM((_BPW,), jnp.int32),
            pltpu.VMEM((_BPW,), jnp.int32),
            pltpu.VMEM((_L, 2, 8, 128), jnp.float32),
            pltpu.VMEM((_L, 2, 8, 128), jnp.float32),
            pltpu.VMEM((_BPW, 128), jnp.float32),
            pltpu.VMEM((_BPW, 128), jnp.float32),
            pltpu.SemaphoreType.DMA,
            pltpu.SemaphoreType.DMA,
        ],
    )
    def gather(cw_hbm, xw_hbm, vt_hbm, ut_hbm, outv_hbm, outu_hbm,
               idx_c, idx_x, buf0, buf1, out_c, out_x, sem0, sem1):
        wid = lax.axis_index("s") * _NC + lax.axis_index("c")
        base = wid * _BPW
        pltpu.sync_copy(cw_hbm.at[pl.ds(base, _BPW)], idx_c)
        pltpu.sync_copy(xw_hbm.at[pl.ds(base, _BPW)], idx_x)

        lanes = lax.iota(jnp.int32, _L)
        bufs = (buf0, buf1)
        sems = (sem0, sem1)

        steps = [(vt_hbm, idx_c, out_c, g) for g in range(_GRP)]
        steps += [(ut_hbm, idx_x, out_x, g) for g in range(_GRP)]

        def issue(step, slot):
            table_hbm, idx, _, g = step
            gran = idx[pl.ds(g * _L, _L)] >> 7
            cps = []
            for k in range(_L):
                c = jnp.max(jnp.where(lanes == k, gran, 0))
                start = pl.multiple_of(c * 128, 128)
                cps.append(pltpu.async_copy(
                    table_hbm.at[:, :, pl.ds(start, 128)],
                    bufs[slot].at[k], sems[slot]))
            return cps

        def extract(step, slot, cps):
            _, idx, out, g = step
            for cp in cps:
                cp.wait()
            lvec = idx[pl.ds(g * _L, _L)] & 127
            rowv = g * _L + lanes
            for e in range(EMBED):
                val = plsc.load_gather(
                    bufs[slot],
                    [lanes, jnp.full((_L,), e // 8, jnp.int32),
                     jnp.full((_L,), e % 8, jnp.int32), lvec])
                plsc.store_scatter(
                    out, [rowv, jnp.full((_L,), e, jnp.int32)], val)

        pending = issue(steps[0], 0)
        for s in range(len(steps)):
            nxt = None
            if s + 1 < len(steps):
                nxt = issue(steps[s + 1], (s + 1) % 2)
            extract(steps[s], s % 2, pending)
            pending = nxt

        pltpu.sync_copy(out_c, outv_hbm.at[pl.ds(base, _BPW)])
        pltpu.sync_copy(out_x, outu_hbm.at[pl.ds(base, _BPW)])

    return gather


_ROW_TILE = 512


def _score_softmax_body(c_ref, x_ref, o_ref):
    scores = lax.dot_general(
        c_ref[:, :EMBED], x_ref[:, :EMBED],
        dimension_numbers=(((1,), (1,)), ((), ())),
        preferred_element_type=jnp.float32,
    )
    m = jnp.max(scores, axis=1, keepdims=True)
    e = jnp.exp(scores - m)
    s = jnp.sum(e, axis=1, keepdims=True)
    o_ref[...] = (scores - m) - jnp.log(s)


def kernel(center_words, context_words, embedding_v, embedding_u):
    vt = embedding_v.T.reshape(2, 8, VOCAB)
    ut = embedding_u.T.reshape(2, 8, VOCAB)
    center_embed, context_embed = _sc_gather_kernel()(
        center_words.astype(jnp.int32), context_words.astype(jnp.int32),
        vt, ut)

    log_probs = pl.pallas_call(
        _score_softmax_body,
        grid=(BATCH // _ROW_TILE,),
        in_specs=[
            pl.BlockSpec((_ROW_TILE, 128), lambda i: (i, 0)),
            pl.BlockSpec((BATCH, 128), lambda i: (0, 0)),
        ],
        out_specs=pl.BlockSpec((_ROW_TILE, BATCH), lambda i: (i, 0)),
        out_shape=jax.ShapeDtypeStruct((BATCH, BATCH), jnp.float32),
    )(center_embed, context_embed)
    return log_probs
